# baseline (speedup 1.0000x reference)
import functools

import jax
import jax.numpy as jnp
from jax import lax
from jax.experimental import pallas as pl
from jax.experimental.pallas import tpu as pltpu
from jax.experimental.pallas import tpu_sc as plsc

B = 16384
DIM = 64
NENT = 1000001
FLAT = 3 * B
NW = 32
ROWS_W = FLAT // NW
CH = 128
NCH = ROWS_W // CH
BE = 4096
GT = 245
NPAD = GT * BE


def _tc_transpose_table(ent_t):
    def body(x_ref, o_ref):
        eye = (lax.broadcasted_iota(jnp.int32, (DIM, DIM), 0)
               == lax.broadcasted_iota(jnp.int32, (DIM, DIM), 1)
               ).astype(jnp.float32)
        t = lax.dot_general(x_ref[...], eye, (((0,), (0,)), ((), ())),
                            preferred_element_type=jnp.float32)
        o_ref[:, 0:DIM] = t
        o_ref[:, DIM:2 * DIM] = t

    return pl.pallas_call(
        body,
        grid=(GT,),
        in_specs=[pl.BlockSpec((DIM, BE), lambda i: (0, i))],
        out_specs=pl.BlockSpec((BE, 2 * DIM), lambda i: (i, 0)),
        out_shape=jax.ShapeDtypeStruct((NPAD, 2 * DIM), jnp.float32),
    )(ent_t)


def _sc_gather_combine(ent_conv, rel_emb, sub_i, obj_i, rel_i):
    mesh = plsc.VectorSubcoreMesh(core_axis_name="c", subcore_axis_name="s")

    @functools.partial(
        pl.kernel,
        mesh=mesh,
        out_type=jax.ShapeDtypeStruct((FLAT, DIM), jnp.float32),
        scratch_types=[
            pltpu.VMEM((NCH, CH), jnp.int32),
            pltpu.VMEM((NCH, CH), jnp.int32),
            pltpu.VMEM((NCH, CH), jnp.int32),
            pltpu.VMEM((CH, 2 * DIM), jnp.float32),
            pltpu.VMEM((CH, 2 * DIM), jnp.float32),
            pltpu.VMEM((CH, DIM), jnp.float32),
            pltpu.SemaphoreType.DMA,
            pltpu.SemaphoreType.DMA,
            pltpu.SemaphoreType.DMA,
        ],
        compiler_params=pltpu.CompilerParams(use_tc_tiling_on_sc=False),
    )
    def k(ent_hbm, rel_hbm, sub_hbm, obj_hbm, reli_hbm, out_hbm,
          idx_s, idx_o, idx_r, buf_s, buf_o, buf_r, sem_s, sem_o, sem_r):
        wid = lax.axis_index("s") * 2 + lax.axis_index("c")
        blk0 = wid * NCH
        pltpu.sync_copy(sub_hbm.at[wid], idx_s)
        pltpu.sync_copy(obj_hbm.at[wid], idx_o)
        pltpu.sync_copy(reli_hbm.at[wid], idx_r)

        def chunk(j, carry):
            cs = pltpu.async_copy(ent_hbm.at[idx_s.at[j]], buf_s, sem_s)
            co = pltpu.async_copy(ent_hbm.at[idx_o.at[j]], buf_o, sem_o)
            cr = pltpu.async_copy(rel_hbm.at[idx_r.at[j]], buf_r, sem_r)
            cs.wait()
            co.wait()
            cr.wait()

            def row(rr, c2):
                for c4 in range(DIM // 16):
                    sl = pl.ds(c4 * 16, 16)
                    buf_r[rr, sl] = buf_s[rr, sl] + buf_r[rr, sl] - buf_o[rr, sl]
                return c2
            lax.fori_loop(0, CH, row, 0)
            pltpu.sync_copy(buf_r, out_hbm.at[pl.ds((blk0 + j) * CH, CH)])
            return carry
        lax.fori_loop(0, NCH, chunk, 0)

    return k(ent_conv, rel_emb, sub_i, obj_i, rel_i)


def _tc_score(d, W, b2):
    BLK = 2048

    def body(d_ref, w_ref, b_ref, o_ref):
        y = lax.dot_general(d_ref[...], w_ref[...],
                            (((1,), (1,)), ((), ())),
                            preferred_element_type=jnp.float32)
        y = y + b_ref[...]
        o_ref[...] = jnp.sum(y * y, axis=1, keepdims=True)

    return pl.pallas_call(
        body,
        grid=(B // BLK,),
        in_specs=[
            pl.BlockSpec((BLK, 3 * DIM), lambda i: (i, 0)),
            pl.BlockSpec((DIM, 3 * DIM), lambda i: (0, 0)),
            pl.BlockSpec((1, DIM), lambda i: (0, 0)),
        ],
        out_specs=pl.BlockSpec((BLK, 1), lambda i: (i, 0)),
        out_shape=jax.ShapeDtypeStruct((B, 1), jnp.float32),
    )(d, W, b2)


def kernel(subjects, objects, relations, ent_emb, rel_emb, W, b):
    ent_conv = _tc_transpose_table(jnp.transpose(ent_emb))
    sub_i = subjects.reshape(NW, NCH, CH)
    obj_i = objects.reshape(NW, NCH, CH)
    rel_i = relations.reshape(NW, NCH, CH)
    d = _sc_gather_combine(ent_conv, rel_emb, sub_i, obj_i, rel_i)
    return _tc_score(d.reshape(B, 3 * DIM), W, b.reshape(1, DIM))


# speedup vs baseline: 1.2023x; 1.2023x over previous
import functools

import jax
import jax.numpy as jnp
from jax import lax
from jax.experimental import pallas as pl
from jax.experimental.pallas import tpu as pltpu
from jax.experimental.pallas import tpu_sc as plsc

B = 16384
DIM = 64
NENT = 1000001
FLAT = 3 * B
NW = 32
ROWS_W = FLAT // NW
CH = 128
NCH = ROWS_W // CH
BE = 2048
GT = 245
H = GT * BE


def _tc_fold_table(ent_t):
    def body(xl_ref, xh_ref, o_ref):
        o_ref[...] = jnp.concatenate([xl_ref[...].T, xh_ref[...].T], axis=1)

    return pl.pallas_call(
        body,
        grid=(GT,),
        in_specs=[
            pl.BlockSpec((DIM, BE), lambda i: (0, i)),
            pl.BlockSpec((DIM, BE),
                         lambda i: (0, jnp.minimum(i + GT, (NENT - 1) // BE))),
        ],
        out_specs=pl.BlockSpec((BE, 2 * DIM), lambda i: (i, 0)),
        out_shape=jax.ShapeDtypeStruct((H, 2 * DIM), jnp.float32),
    )(ent_t, ent_t)


def _sc_gather(ent_fold, rel_emb, sub_i, obj_i, rel_i):
    mesh = plsc.VectorSubcoreMesh(core_axis_name="c", subcore_axis_name="s")

    @functools.partial(
        pl.kernel,
        mesh=mesh,
        out_type=[
            jax.ShapeDtypeStruct((FLAT, 2 * DIM), jnp.float32),
            jax.ShapeDtypeStruct((FLAT, 2 * DIM), jnp.float32),
            jax.ShapeDtypeStruct((FLAT, DIM), jnp.float32),
        ],
        scratch_types=[
            pltpu.VMEM((NCH, CH), jnp.int32),
            pltpu.VMEM((NCH, CH), jnp.int32),
            pltpu.VMEM((NCH, CH), jnp.int32),
            pltpu.VMEM((CH, 2 * DIM), jnp.float32),
            pltpu.VMEM((CH, 2 * DIM), jnp.float32),
            pltpu.VMEM((CH, DIM), jnp.float32),
            pltpu.SemaphoreType.DMA,
            pltpu.SemaphoreType.DMA,
            pltpu.SemaphoreType.DMA,
        ],
        compiler_params=pltpu.CompilerParams(use_tc_tiling_on_sc=False),
    )
    def k(ent_hbm, rel_hbm, sub_hbm, obj_hbm, reli_hbm,
          outs_hbm, outo_hbm, outr_hbm,
          idx_s, idx_o, idx_r, buf_s, buf_o, buf_r, sem_s, sem_o, sem_r):
        wid = lax.axis_index("s") * 2 + lax.axis_index("c")
        blk0 = wid * NCH
        pltpu.sync_copy(sub_hbm.at[wid], idx_s)
        pltpu.sync_copy(obj_hbm.at[wid], idx_o)
        pltpu.sync_copy(reli_hbm.at[wid], idx_r)

        def chunk(j, carry):
            cs = pltpu.async_copy(ent_hbm.at[idx_s.at[j]], buf_s, sem_s)
            co = pltpu.async_copy(ent_hbm.at[idx_o.at[j]], buf_o, sem_o)
            cr = pltpu.async_copy(rel_hbm.at[idx_r.at[j]], buf_r, sem_r)
            rows = pl.ds((blk0 + j) * CH, CH)
            cs.wait()
            pltpu.sync_copy(buf_s, outs_hbm.at[rows])
            co.wait()
            pltpu.sync_copy(buf_o, outo_hbm.at[rows])
            cr.wait()
            pltpu.sync_copy(buf_r, outr_hbm.at[rows])
            return carry
        lax.fori_loop(0, NCH, chunk, 0)

    return k(ent_fold, rel_emb, sub_i, obj_i, rel_i)


def _tc_score(s128, o128, r64, ps, po, Wdup, Wt, b2):
    BLK = 2048
    NB = B // BLK

    def body(s0, s1, s2, o0, o1, o2, r0, r1, r2, ps_ref, po_ref,
             wd_ref, wt_ref, b_ref, out_ref):
        halfbit = (lax.broadcasted_iota(jnp.int32, (BLK, 2 * DIM), 1)
                   >> 6) & 1
        y = jnp.broadcast_to(b_ref[...], (BLK, DIM))
        for k, (s_ref, o_ref, r_ref) in enumerate(
                ((s0, o0, r0), (s1, o1, r1), (s2, o2, r2))):
            m_s = (jnp.broadcast_to(ps_ref[:, k:k + 1], (BLK, 2 * DIM))
                   == halfbit).astype(jnp.float32)
            m_o = (jnp.broadcast_to(po_ref[:, k:k + 1], (BLK, 2 * DIM))
                   == halfbit).astype(jnp.float32)
            h = s_ref[...] * m_s - o_ref[...] * m_o
            y = y + lax.dot_general(
                h, wd_ref[2 * DIM * k:2 * DIM * (k + 1), :],
                (((1,), (0,)), ((), ())),
                preferred_element_type=jnp.float32)
            y = y + lax.dot_general(
                r_ref[...], wt_ref[DIM * k:DIM * (k + 1), :],
                (((1,), (0,)), ((), ())),
                preferred_element_type=jnp.float32)
        out_ref[...] = jnp.sum(y * y, axis=1, keepdims=True)

    def slot_spec(width, k):
        return pl.BlockSpec((BLK, width), lambda i, _k=k: (_k * NB + i, 0))

    return pl.pallas_call(
        body,
        grid=(NB,),
        in_specs=[
            slot_spec(2 * DIM, 0), slot_spec(2 * DIM, 1),
            slot_spec(2 * DIM, 2),
            slot_spec(2 * DIM, 0), slot_spec(2 * DIM, 1),
            slot_spec(2 * DIM, 2),
            slot_spec(DIM, 0), slot_spec(DIM, 1), slot_spec(DIM, 2),
            pl.BlockSpec((BLK, 3), lambda i: (i, 0)),
            pl.BlockSpec((BLK, 3), lambda i: (i, 0)),
            pl.BlockSpec((6 * DIM, DIM), lambda i: (0, 0)),
            pl.BlockSpec((3 * DIM, DIM), lambda i: (0, 0)),
            pl.BlockSpec((1, DIM), lambda i: (0, 0)),
        ],
        out_specs=pl.BlockSpec((BLK, 1), lambda i: (i, 0)),
        out_shape=jax.ShapeDtypeStruct((B, 1), jnp.float32),
    )(s128, s128, s128, o128, o128, o128, r64, r64, r64,
      ps, po, Wdup, Wt, b2)


def kernel(subjects, objects, relations, ent_emb, rel_emb, W, b):
    ent_fold = _tc_fold_table(jnp.transpose(ent_emb))
    srow = jnp.where(subjects < H, subjects, subjects - H)
    orow = jnp.where(objects < H, objects, objects - H)
    ps = (subjects >= H).astype(jnp.int32)
    po = (objects >= H).astype(jnp.int32)
    s128, o128, r64 = _sc_gather(
        ent_fold, rel_emb,
        srow.T.reshape(NW, NCH, CH), orow.T.reshape(NW, NCH, CH),
        relations.T.reshape(NW, NCH, CH))
    Wt = W.T
    Wdup = jnp.concatenate(
        [Wt[k * DIM:(k + 1) * DIM] for k in range(3) for _ in range(2)],
        axis=0)
    return _tc_score(s128, o128, r64, ps, po, Wdup, Wt, b.reshape(1, DIM))
